# baseline (speedup 1.0000x reference)
import functools

import jax
import jax.numpy as jnp
from jax import lax
from jax.experimental import pallas as pl
from jax.experimental.pallas import tpu as pltpu
from jax.experimental.pallas import tpu_sc as plsc

N = 10000
E = 320000
TEMP = 0.2
MAX_NUMS = (512, 64)

NC, NS = 2, 16
NW = NC * NS
EPW = 10240
EPAD = NW * EPW
NPAD = 10112
ROWS_PER_TILE = NPAD // NS


def _seg_sum_kernel(n_chunks, W, tiled):
    B = 128 if tiled else 64
    NBLK = EPW // B
    mesh = plsc.VectorSubcoreMesh(
        core_axis_name="c", subcore_axis_name="s", num_cores=NC, num_subcores=NS)

    @functools.partial(
        pl.kernel,
        out_type=jax.ShapeDtypeStruct((NC, n_chunks, NPAD, W), jnp.float32),
        mesh=mesh,
        scratch_types=[pltpu.VMEM((EPW,), jnp.int32)]
        + [pltpu.VMEM((B,), jnp.int32)] * 4
        + [pltpu.VMEM((B, W), jnp.float32)] * 2
        + [pltpu.VMEM_SHARED((NPAD, W), jnp.float32)]
        + [pltpu.SemaphoreType.DMA] * 7,
        compiler_params=pltpu.CompilerParams(use_tc_tiling_on_sc=tiled),
    )
    def k(tab_h, src_h, dst_h, zrow_h, out_h, *refs):
        srcall = refs[0]
        dstb = refs[1:5]
        ring = refs[5:7]
        acc = refs[7]
        dsems, gsems, ssem = refs[8:12], refs[12:14], refs[14]
        cid = lax.axis_index("c")
        sid = lax.axis_index("s")
        wid = cid * NS + sid

        def dst_fetch(b, q):
            pltpu.async_copy(dst_h.at[pl.ds(wid * EPW + b * B, B)], dstb[q], dsems[q])

        def gather(b, s):
            pltpu.async_copy(tab_h.at[0].at[srcall.at[pl.ds(b * B, B)]], ring[s], gsems[s])

        pltpu.sync_copy(src_h.at[pl.ds(wid * EPW, EPW)], srcall)
        for c in range(n_chunks):
            tab_c = tab_h.at[c]
            pltpu.sync_copy(zrow_h, acc.at[pl.ds(sid * ROWS_PER_TILE, ROWS_PER_TILE)])
            for q in range(4):
                dst_fetch(q, q)
            for s in range(2):
                pltpu.async_copy(tab_c.at[srcall.at[pl.ds(s * B, B)]], ring[s], gsems[s])
            plsc.subcore_barrier()

            def step(b, s, q):
                pltpu.make_async_copy(tab_c.at[srcall.at[pl.ds(0, B)]], ring[s], gsems[s]).wait()
                pltpu.make_async_copy(dst_h.at[pl.ds(wid * EPW, B)], dstb[q], dsems[q]).wait()
                pltpu.async_copy(ring[s], acc.at[dstb[q]], ssem, add=True).wait()

                @pl.when(b + 4 < NBLK)
                def _():
                    dst_fetch(b + 4, q)

                @pl.when(b + 2 < NBLK)
                def _():
                    pltpu.async_copy(
                        tab_c.at[srcall.at[pl.ds((b + 2) * B, B)]], ring[s], gsems[s])

            def quad(i, _):
                b0 = 4 * i
                step(b0, 0, 0)
                step(b0 + 1, 1, 1)
                step(b0 + 2, 0, 2)
                step(b0 + 3, 1, 3)
                return 0

            lax.fori_loop(0, NBLK // 4, quad, 0)
            plsc.subcore_barrier()
            pltpu.sync_copy(
                acc.at[pl.ds(sid * ROWS_PER_TILE, ROWS_PER_TILE)],
                out_h.at[cid].at[c].at[pl.ds(sid * ROWS_PER_TILE, ROWS_PER_TILE)])
            plsc.subcore_barrier()

    return k


BLK = 2000
G = N // BLK


def _pad2(w, rows, cols):
    return jnp.pad(w, ((0, rows - w.shape[0]), (0, cols - w.shape[1])))


def _pad_lin(Wm, b, s, rows, cols):
    return _pad2(Wm, rows, cols), jnp.pad(b, (0, cols - b.shape[0]))[None, :], s.reshape(1, 1)


def _ll_block(xp, W, b, se, nonlin):
    if nonlin:
        xp = jax.nn.relu(xp)
    y = jax.lax.dot_general(xp, W, (((1,), (0,)), ((), ())),
                            preferred_element_type=jnp.float32) + b
    t = jax.nn.sigmoid(y[:, :1]) * jnp.exp(se[0, 0]) + 1.1
    xn2 = jnp.sum(y * y, axis=1, keepdims=True) - y[:, :1] * y[:, :1]
    sc = (t * t - 1.0) / jnp.clip(xn2, 1e-8, None)
    col = jax.lax.broadcasted_iota(jnp.int32, y.shape, 1)
    return jnp.where(col == 0, t, y * jnp.sqrt(sc))


def _norm_block(s):
    inner = jnp.sum(s * s, axis=1, keepdims=True) - 2.0 * s[:, :1] * s[:, :1]
    return s / jnp.sqrt(jnp.clip(jnp.abs(inner), 1e-8, None))


def _rspec(w):
    return pl.BlockSpec(w.shape, lambda i: (0,) * w.ndim)


def _k1_body(x_ref, W_ref, b_ref, s_ref, o_ref):
    xb = x_ref[...]
    nrm = jnp.clip(jnp.sqrt(jnp.sum(xb * xb, axis=1, keepdims=True)), 1e-8, None)
    ep, en = jnp.exp(nrm), jnp.exp(-nrm)
    cosh_n, sinh_n = 0.5 * (ep + en), 0.5 * (ep - en)
    h0 = jnp.concatenate(
        [cosh_n, sinh_n * xb / nrm, jnp.zeros((xb.shape[0], 15), jnp.float32)],
        axis=1)
    o_ref[...] = _ll_block(h0, W_ref[...], b_ref[...], s_ref[...], True)


def _k2_body(sab_ref, W_ref, b_ref, s_ref, o_ref):
    s_ = sab_ref[0, 0] + sab_ref[1, 0]
    o_ref[...] = _ll_block(_norm_block(s_), W_ref[...], b_ref[...], s_ref[...], True)


def _k3_body(sab_ref, Wp_ref, bp_ref, sp_ref, Wa_ref, ba_ref, sa_ref, z_ref, ll3_ref):
    s_ = sab_ref[0, 0] + sab_ref[1, 0]
    hn = _norm_block(s_)
    z_ref[...] = hn[:, :129]
    hp = _ll_block(hn, Wp_ref[...], bp_ref[...], sp_ref[...], False)
    ll3_ref[...] = _ll_block(hp, Wa_ref[...], ba_ref[...], sa_ref[...], True)


def _k4_body(sab_ref, ass_ref):
    s_ = sab_ref[0] + sab_ref[1]
    row = jnp.concatenate([s_[0], s_[1], s_[2], s_[3]], axis=1)
    nrm = _norm_block(row)
    col = jax.lax.broadcasted_iota(jnp.int32, nrm.shape, 1)
    logits = jnp.where((col >= 1) & (col <= 512), nrm / TEMP, -jnp.inf)
    m = jnp.max(logits, axis=1, keepdims=True)
    e = jnp.exp(logits - m)
    sm = e / jnp.sum(e, axis=1, keepdims=True)
    ass_ref[...] = sm[:, 1:513]


def _k5_body(ass_ref, z_ref, o_ref):
    k = pl.program_id(0)

    @pl.when(k == 0)
    def _():
        o_ref[...] = jnp.zeros_like(o_ref)

    o_ref[...] += jax.lax.dot_general(
        ass_ref[...], z_ref[...], (((0,), (0,)), ((), ())),
        preferred_element_type=jnp.float32)

    @pl.when(k == G - 1)
    def _():
        o_ref[...] = _norm_block(o_ref[...])


def _k6_body(ass_ref, sab_ref, o_ref):
    k = pl.program_id(0)

    @pl.when(k == 0)
    def _():
        o_ref[...] = jnp.zeros_like(o_ref)

    s_ = sab_ref[0] + sab_ref[1]
    aass = jnp.concatenate([s_[0], s_[1], s_[2], s_[3]], axis=1)
    o_ref[...] += jax.lax.dot_general(
        ass_ref[...], aass, (((0,), (0,)), ((), ())),
        preferred_element_type=jnp.float32)

    @pl.when(k == G - 1)
    def _():
        v = o_ref[...]
        i = jax.lax.broadcasted_iota(jnp.int32, v.shape, 0)
        j = jax.lax.broadcasted_iota(jnp.int32, v.shape, 1)
        o_ref[...] = jax.nn.sigmoid(jnp.where(i == j, 0.0, v) / TEMP)


def _k7_body(z2_ref, adj2_ref, Wp_ref, bp_ref, sp_ref, Wa_ref, ba_ref, sa_ref,
             z1_ref, root_ref, ass2_ref, adj1_ref):
    z2 = z2_ref[...]
    z2p = jnp.concatenate([z2, jnp.zeros((z2.shape[0], 15), jnp.float32)], axis=1)
    adj2 = adj2_ref[...]
    h2 = _ll_block(z2p, Wp_ref[...], bp_ref[...], sp_ref[...], False)
    h2b = _ll_block(h2, Wa_ref[...], ba_ref[...], sa_ref[...], True)
    agg2 = _norm_block(jax.lax.dot_general(
        adj2, h2b, (((1,), (0,)), ((), ())), preferred_element_type=jnp.float32))
    col = jax.lax.broadcasted_iota(jnp.int32, agg2.shape, 1)
    logits = jnp.where((col >= 1) & (col <= 64), agg2 / TEMP, -jnp.inf)
    m = jnp.max(logits, axis=1, keepdims=True)
    e = jnp.exp(logits - m)
    ass2 = (e / jnp.sum(e, axis=1, keepdims=True))[:, 1:65]
    ass2_ref[...] = ass2
    z1 = _norm_block(jax.lax.dot_general(
        ass2, z2p, (((0,), (0,)), ((), ())), preferred_element_type=jnp.float32))
    z1_ref[...] = z1[:, :129]
    root_ref[...] = _norm_block(jnp.sum(z1, axis=0, keepdims=True))[:, :129]
    a2 = jax.lax.dot_general(adj2, ass2, (((1,), (0,)), ((), ())),
                             preferred_element_type=jnp.float32)
    adj1v = jax.lax.dot_general(ass2, a2, (((0,), (0,)), ((), ())),
                                preferred_element_type=jnp.float32)
    i = jax.lax.broadcasted_iota(jnp.int32, adj1v.shape, 0)
    j = jax.lax.broadcasted_iota(jnp.int32, adj1v.shape, 1)
    adj1_ref[...] = jax.nn.sigmoid(jnp.where(i == j, 0.0, adj1v) / TEMP)


def _row_spec(w):
    return pl.BlockSpec((BLK,) + w[1:], lambda i: (i,) + (0,) * len(w[1:]))


def _part_spec(n_chunks, w):
    return pl.BlockSpec((2, n_chunks, BLK, w), lambda i: (0, 0, i, 0))


def kernel(x, edge_index, W1, b1, s1, W2, b2, s2, Wp1, bp1, sp1, Wa1, ba1, sa1, Wp2, bp2, sp2, Wa2, ba2, sa2):
    npad_e = EPW - E // NW
    src1d = jnp.concatenate(
        [edge_index[0].reshape(NW, E // NW),
         jnp.zeros((NW, npad_e), jnp.int32)], axis=1).reshape(-1)
    dst1d = jnp.concatenate(
        [edge_index[1].reshape(NW, E // NW),
         jnp.full((NW, npad_e), NPAD - 1, jnp.int32)], axis=1).reshape(-1)
    zrows = {128: jnp.zeros((ROWS_PER_TILE, 128), jnp.float32),
             144: jnp.zeros((ROWS_PER_TILE, 144), jnp.float32)}

    W1p, b1p, s1p = _pad_lin(W1, b1, s1, 144, 144)
    W2p, b2p, s2p = _pad_lin(W2, b2, s2, 144, 144)
    Wp1p, bp1p, sp1p = _pad_lin(Wp1, bp1, sp1, 144, 144)
    Wa1p, ba1p, sa1p = _pad_lin(Wa1, ba1, sa1, 144, 576)
    Wp2p, bp2p, sp2p = _pad_lin(Wp2, bp2, sp2, 144, 144)
    Wa2p, ba2p, sa2p = _pad_lin(Wa2, ba2, sa2, 144, 80)

    ll1 = pl.pallas_call(
        _k1_body, grid=(G,),
        in_specs=[_row_spec((N, 128)), _rspec(W1p), _rspec(b1p), _rspec(s1p)],
        out_specs=_row_spec((N, 144)),
        out_shape=jax.ShapeDtypeStruct((N, 144), jnp.float32),
    )(x, W1p, b1p, s1p)
    s1_parts = _seg_sum_kernel(1, 144, False)(ll1[None], src1d, dst1d, zrows[144])

    ll2 = pl.pallas_call(
        _k2_body, grid=(G,),
        in_specs=[_part_spec(1, 144), _rspec(W2p), _rspec(b2p), _rspec(s2p)],
        out_specs=_row_spec((N, 144)),
        out_shape=jax.ShapeDtypeStruct((N, 144), jnp.float32),
    )(s1_parts, W2p, b2p, s2p)
    s2_parts = _seg_sum_kernel(1, 144, False)(ll2[None], src1d, dst1d, zrows[144])

    z, ll3 = pl.pallas_call(
        _k3_body, grid=(G,),
        in_specs=[_part_spec(1, 144), _rspec(Wp1p), _rspec(bp1p), _rspec(sp1p),
                  _rspec(Wa1p), _rspec(ba1p), _rspec(sa1p)],
        out_specs=[_row_spec((N, 129)), _row_spec((N, 576))],
        out_shape=[jax.ShapeDtypeStruct((N, 129), jnp.float32),
                   jax.ShapeDtypeStruct((N, 576), jnp.float32)],
    )(s2_parts, Wp1p, bp1p, sp1p, Wa1p, ba1p, sa1p)
    tab3 = ll3.reshape(N, 4, 144).transpose(1, 0, 2)
    s3_parts = _seg_sum_kernel(4, 144, False)(tab3, src1d, dst1d, zrows[144])

    ass3 = pl.pallas_call(
        _k4_body, grid=(G,),
        in_specs=[_part_spec(4, 144)],
        out_specs=_row_spec((N, 512)),
        out_shape=jax.ShapeDtypeStruct((N, 512), jnp.float32),
    )(s3_parts)
    tab4 = ass3.reshape(N, 4, 128).transpose(1, 0, 2)
    s4_parts = _seg_sum_kernel(4, 128, True)(tab4, src1d, dst1d, zrows[128])

    z2 = pl.pallas_call(
        _k5_body, grid=(G,),
        in_specs=[_row_spec((N, 512)), _row_spec((N, 129))],
        out_specs=pl.BlockSpec((512, 129), lambda i: (0, 0)),
        out_shape=jax.ShapeDtypeStruct((512, 129), jnp.float32),
    )(ass3, z)

    adj2 = pl.pallas_call(
        _k6_body, grid=(G,),
        in_specs=[_row_spec((N, 512)), _part_spec(4, 128)],
        out_specs=pl.BlockSpec((512, 512), lambda i: (0, 0)),
        out_shape=jax.ShapeDtypeStruct((512, 512), jnp.float32),
    )(ass3, s4_parts)

    z1, root, ass2, adj1 = pl.pallas_call(
        _k7_body, grid=(1,),
        in_specs=[pl.BlockSpec((512, 129), lambda i: (0, 0)),
                  pl.BlockSpec((512, 512), lambda i: (0, 0)),
                  _rspec(Wp2p), _rspec(bp2p), _rspec(sp2p),
                  _rspec(Wa2p), _rspec(ba2p), _rspec(sa2p)],
        out_specs=[pl.BlockSpec((64, 129), lambda i: (0, 0)),
                   pl.BlockSpec((1, 129), lambda i: (0, 0)),
                   pl.BlockSpec((512, 64), lambda i: (0, 0)),
                   pl.BlockSpec((64, 64), lambda i: (0, 0))],
        out_shape=[jax.ShapeDtypeStruct((64, 129), jnp.float32),
                   jax.ShapeDtypeStruct((1, 129), jnp.float32),
                   jax.ShapeDtypeStruct((512, 64), jnp.float32),
                   jax.ShapeDtypeStruct((64, 64), jnp.float32)],
    )(z2, adj2, Wp2p, bp2p, sp2p, Wa2p, ba2p, sa2p)

    ass1 = jnp.ones((64, 1), dtype=z1.dtype)
    return (z, z2, z1, root, ass3, ass2, ass1, adj2, adj1)


# speedup vs baseline: 1.0350x; 1.0350x over previous
import functools

import jax
import jax.numpy as jnp
from jax import lax
from jax.experimental import pallas as pl
from jax.experimental.pallas import tpu as pltpu
from jax.experimental.pallas import tpu_sc as plsc

N = 10000
E = 320000
TEMP = 0.2
MAX_NUMS = (512, 64)

NC, NS = 2, 16
NW = NC * NS
EPW = 10240
EPAD = NW * EPW
NPAD = 10112
ROWS_PER_TILE = NPAD // NS


def _seg_sum_kernel(n_chunks, W, tiled):
    B = 128 if tiled else 64
    NBLK = EPW // B
    mesh = plsc.VectorSubcoreMesh(
        core_axis_name="c", subcore_axis_name="s", num_cores=NC, num_subcores=NS)

    @functools.partial(
        pl.kernel,
        out_type=jax.ShapeDtypeStruct((NC, n_chunks, NPAD, W), jnp.float32),
        mesh=mesh,
        scratch_types=[pltpu.VMEM((EPW,), jnp.int32)]
        + [pltpu.VMEM((B,), jnp.int32)] * 4
        + [pltpu.VMEM((B, W), jnp.float32)] * 2
        + [pltpu.VMEM_SHARED((NPAD, W), jnp.float32)]
        + [pltpu.SemaphoreType.DMA] * 7,
        compiler_params=pltpu.CompilerParams(use_tc_tiling_on_sc=tiled),
    )
    def k(tab_h, src_h, dst_h, zrow_h, out_h, *refs):
        srcall = refs[0]
        dstb = refs[1:5]
        ring = refs[5:7]
        acc = refs[7]
        dsems, gsems, ssem = refs[8:12], refs[12:14], refs[14]
        cid = lax.axis_index("c")
        sid = lax.axis_index("s")
        wid = cid * NS + sid

        def dst_fetch(b, q):
            pltpu.async_copy(dst_h.at[pl.ds(wid * EPW + b * B, B)], dstb[q], dsems[q])

        pltpu.sync_copy(src_h.at[pl.ds(wid * EPW, EPW)], srcall)
        for c in range(n_chunks):
            tab_c = tab_h.at[c]
            pltpu.sync_copy(zrow_h, acc.at[pl.ds(sid * ROWS_PER_TILE, ROWS_PER_TILE)])
            for q in range(4):
                dst_fetch(q, q)
            for s in range(2):
                pltpu.async_copy(tab_c.at[srcall.at[pl.ds(s * B, B)]], ring[s], gsems[s])
            plsc.subcore_barrier()

            def step(b, s, q):
                pltpu.make_async_copy(tab_c.at[srcall.at[pl.ds(0, B)]], ring[s], gsems[s]).wait()
                pltpu.make_async_copy(dst_h.at[pl.ds(wid * EPW, B)], dstb[q], dsems[q]).wait()
                pltpu.async_copy(ring[s], acc.at[dstb[q]], ssem, add=True).wait()

                @pl.when(b + 4 < NBLK)
                def _():
                    dst_fetch(b + 4, q)

                @pl.when(b + 2 < NBLK)
                def _():
                    pltpu.async_copy(
                        tab_c.at[srcall.at[pl.ds((b + 2) * B, B)]], ring[s], gsems[s])

            def quad(i, _):
                b0 = 4 * i
                step(b0, 0, 0)
                step(b0 + 1, 1, 1)
                step(b0 + 2, 0, 2)
                step(b0 + 3, 1, 3)
                return 0

            lax.fori_loop(0, NBLK // 4, quad, 0)
            plsc.subcore_barrier()
            pltpu.sync_copy(
                acc.at[pl.ds(sid * ROWS_PER_TILE, ROWS_PER_TILE)],
                out_h.at[cid].at[c].at[pl.ds(sid * ROWS_PER_TILE, ROWS_PER_TILE)])
            plsc.subcore_barrier()

    return k


BLK = 2000
G = N // BLK


def _pad2(w, rows, cols):
    return jnp.pad(w, ((0, rows - w.shape[0]), (0, cols - w.shape[1])))


def _pad_lin(Wm, b, s, rows, cols):
    return _pad2(Wm, rows, cols), jnp.pad(b, (0, cols - b.shape[0]))[None, :], s.reshape(1, 1)


def _ll_block(xp, W, b, se, nonlin):
    if nonlin:
        xp = jax.nn.relu(xp)
    y = jax.lax.dot_general(xp, W, (((1,), (0,)), ((), ())),
                            preferred_element_type=jnp.float32) + b
    t = jax.nn.sigmoid(y[:, :1]) * jnp.exp(se[0, 0]) + 1.1
    xn2 = jnp.sum(y * y, axis=1, keepdims=True) - y[:, :1] * y[:, :1]
    sc = (t * t - 1.0) / jnp.clip(xn2, 1e-8, None)
    col = jax.lax.broadcasted_iota(jnp.int32, y.shape, 1)
    return jnp.where(col == 0, t, y * jnp.sqrt(sc))


def _norm_block(s):
    inner = jnp.sum(s * s, axis=1, keepdims=True) - 2.0 * s[:, :1] * s[:, :1]
    return s / jnp.sqrt(jnp.clip(jnp.abs(inner), 1e-8, None))


def _rspec(w):
    return pl.BlockSpec(w.shape, lambda i: (0,) * w.ndim)


def _k1_body(x_ref, W_ref, b_ref, s_ref, o_ref):
    xb = x_ref[...]
    nrm = jnp.clip(jnp.sqrt(jnp.sum(xb * xb, axis=1, keepdims=True)), 1e-8, None)
    ep, en = jnp.exp(nrm), jnp.exp(-nrm)
    cosh_n, sinh_n = 0.5 * (ep + en), 0.5 * (ep - en)
    h0 = jnp.concatenate(
        [cosh_n, sinh_n * xb / nrm, jnp.zeros((xb.shape[0], 15), jnp.float32)],
        axis=1)
    o_ref[...] = _ll_block(h0, W_ref[...], b_ref[...], s_ref[...], True)


def _k2_body(sab_ref, W_ref, b_ref, s_ref, o_ref):
    s_ = sab_ref[0, 0] + sab_ref[1, 0]
    o_ref[...] = _ll_block(_norm_block(s_), W_ref[...], b_ref[...], s_ref[...], True)


def _k3_body(sab_ref, Wp_ref, bp_ref, sp_ref, Wa_ref, ba_ref, sa_ref, z_ref, ll3_ref):
    s_ = sab_ref[0, 0] + sab_ref[1, 0]
    hn = _norm_block(s_)
    z_ref[...] = hn[:, :129]
    hp = _ll_block(hn, Wp_ref[...], bp_ref[...], sp_ref[...], False)
    ll3_ref[...] = _ll_block(hp, Wa_ref[...], ba_ref[...], sa_ref[...], True)


def _k4_body(sab_ref, ass_ref):
    s_ = sab_ref[0] + sab_ref[1]
    row = jnp.concatenate([s_[0], s_[1], s_[2], s_[3]], axis=1)
    nrm = _norm_block(row)
    col = jax.lax.broadcasted_iota(jnp.int32, nrm.shape, 1)
    logits = jnp.where((col >= 1) & (col <= 512), nrm / TEMP, -jnp.inf)
    m = jnp.max(logits, axis=1, keepdims=True)
    e = jnp.exp(logits - m)
    sm = e / jnp.sum(e, axis=1, keepdims=True)
    ass_ref[...] = sm[:, 1:513]


def _k5_body(ass_ref, z_ref, o_ref):
    k = pl.program_id(0)

    @pl.when(k == 0)
    def _():
        o_ref[...] = jnp.zeros_like(o_ref)

    o_ref[...] += jax.lax.dot_general(
        ass_ref[...], z_ref[...], (((0,), (0,)), ((), ())),
        preferred_element_type=jnp.float32)

    @pl.when(k == G - 1)
    def _():
        o_ref[...] = _norm_block(o_ref[...])


def _k6_body(ass_ref, sab_ref, o_ref):
    k = pl.program_id(0)

    @pl.when(k == 0)
    def _():
        o_ref[...] = jnp.zeros_like(o_ref)

    s_ = sab_ref[0] + sab_ref[1]
    aass = jnp.concatenate([s_[0], s_[1], s_[2], s_[3]], axis=1)
    o_ref[...] += jax.lax.dot_general(
        ass_ref[...], aass, (((0,), (0,)), ((), ())),
        preferred_element_type=jnp.float32)

    @pl.when(k == G - 1)
    def _():
        v = o_ref[...]
        i = jax.lax.broadcasted_iota(jnp.int32, v.shape, 0)
        j = jax.lax.broadcasted_iota(jnp.int32, v.shape, 1)
        o_ref[...] = jax.nn.sigmoid(jnp.where(i == j, 0.0, v) / TEMP)


def _k7_body(z2_ref, adj2_ref, Wp_ref, bp_ref, sp_ref, Wa_ref, ba_ref, sa_ref,
             z1_ref, root_ref, ass2_ref, adj1_ref):
    z2 = z2_ref[...]
    z2p = jnp.concatenate([z2, jnp.zeros((z2.shape[0], 15), jnp.float32)], axis=1)
    adj2 = adj2_ref[...]
    h2 = _ll_block(z2p, Wp_ref[...], bp_ref[...], sp_ref[...], False)
    h2b = _ll_block(h2, Wa_ref[...], ba_ref[...], sa_ref[...], True)
    agg2 = _norm_block(jax.lax.dot_general(
        adj2, h2b, (((1,), (0,)), ((), ())), preferred_element_type=jnp.float32))
    col = jax.lax.broadcasted_iota(jnp.int32, agg2.shape, 1)
    logits = jnp.where((col >= 1) & (col <= 64), agg2 / TEMP, -jnp.inf)
    m = jnp.max(logits, axis=1, keepdims=True)
    e = jnp.exp(logits - m)
    ass2 = (e / jnp.sum(e, axis=1, keepdims=True))[:, 1:65]
    ass2_ref[...] = ass2
    z1 = _norm_block(jax.lax.dot_general(
        ass2, z2p, (((0,), (0,)), ((), ())), preferred_element_type=jnp.float32))
    z1_ref[...] = z1[:, :129]
    root_ref[...] = _norm_block(jnp.sum(z1, axis=0, keepdims=True))[:, :129]
    a2 = jax.lax.dot_general(adj2, ass2, (((1,), (0,)), ((), ())),
                             preferred_element_type=jnp.float32)
    adj1v = jax.lax.dot_general(ass2, a2, (((0,), (0,)), ((), ())),
                                preferred_element_type=jnp.float32)
    i = jax.lax.broadcasted_iota(jnp.int32, adj1v.shape, 0)
    j = jax.lax.broadcasted_iota(jnp.int32, adj1v.shape, 1)
    adj1_ref[...] = jax.nn.sigmoid(jnp.where(i == j, 0.0, adj1v) / TEMP)


def _row_spec(w):
    return pl.BlockSpec((BLK,) + w[1:], lambda i: (i,) + (0,) * len(w[1:]))


def _part_spec(n_chunks, w):
    return pl.BlockSpec((2, n_chunks, BLK, w), lambda i: (0, 0, i, 0))


def kernel(x, edge_index, W1, b1, s1, W2, b2, s2, Wp1, bp1, sp1, Wa1, ba1, sa1, Wp2, bp2, sp2, Wa2, ba2, sa2):
    npad_e = EPW - E // NW
    src1d = jnp.concatenate(
        [edge_index[0].reshape(NW, E // NW),
         jnp.zeros((NW, npad_e), jnp.int32)], axis=1).reshape(-1)
    pad_rows = N + jnp.arange(NW, dtype=jnp.int32)[:, None] + jnp.zeros((1, npad_e), jnp.int32)
    dst1d = jnp.concatenate(
        [edge_index[1].reshape(NW, E // NW), pad_rows], axis=1).reshape(-1)
    zrows = {128: jnp.zeros((ROWS_PER_TILE, 128), jnp.float32),
             144: jnp.zeros((ROWS_PER_TILE, 144), jnp.float32)}

    W1p, b1p, s1p = _pad_lin(W1, b1, s1, 144, 144)
    W2p, b2p, s2p = _pad_lin(W2, b2, s2, 144, 144)
    Wp1p, bp1p, sp1p = _pad_lin(Wp1, bp1, sp1, 144, 144)
    Wa1p, ba1p, sa1p = _pad_lin(Wa1, ba1, sa1, 144, 576)
    Wp2p, bp2p, sp2p = _pad_lin(Wp2, bp2, sp2, 144, 144)
    Wa2p, ba2p, sa2p = _pad_lin(Wa2, ba2, sa2, 144, 80)

    ll1 = pl.pallas_call(
        _k1_body, grid=(G,),
        in_specs=[_row_spec((N, 128)), _rspec(W1p), _rspec(b1p), _rspec(s1p)],
        out_specs=_row_spec((N, 144)),
        out_shape=jax.ShapeDtypeStruct((N, 144), jnp.float32),
    )(x, W1p, b1p, s1p)
    s1_parts = _seg_sum_kernel(1, 144, False)(ll1[None], src1d, dst1d, zrows[144])

    ll2 = pl.pallas_call(
        _k2_body, grid=(G,),
        in_specs=[_part_spec(1, 144), _rspec(W2p), _rspec(b2p), _rspec(s2p)],
        out_specs=_row_spec((N, 144)),
        out_shape=jax.ShapeDtypeStruct((N, 144), jnp.float32),
    )(s1_parts, W2p, b2p, s2p)
    s2_parts = _seg_sum_kernel(1, 144, False)(ll2[None], src1d, dst1d, zrows[144])

    z, ll3 = pl.pallas_call(
        _k3_body, grid=(G,),
        in_specs=[_part_spec(1, 144), _rspec(Wp1p), _rspec(bp1p), _rspec(sp1p),
                  _rspec(Wa1p), _rspec(ba1p), _rspec(sa1p)],
        out_specs=[_row_spec((N, 129)), _row_spec((N, 576))],
        out_shape=[jax.ShapeDtypeStruct((N, 129), jnp.float32),
                   jax.ShapeDtypeStruct((N, 576), jnp.float32)],
    )(s2_parts, Wp1p, bp1p, sp1p, Wa1p, ba1p, sa1p)
    tab3 = ll3.reshape(N, 4, 144).transpose(1, 0, 2)
    s3_parts = _seg_sum_kernel(4, 144, False)(tab3, src1d, dst1d, zrows[144])

    ass3 = pl.pallas_call(
        _k4_body, grid=(G,),
        in_specs=[_part_spec(4, 144)],
        out_specs=_row_spec((N, 512)),
        out_shape=jax.ShapeDtypeStruct((N, 512), jnp.float32),
    )(s3_parts)
    tab4 = ass3.reshape(N, 4, 128).transpose(1, 0, 2)
    s4_parts = _seg_sum_kernel(4, 128, True)(tab4, src1d, dst1d, zrows[128])

    z2 = pl.pallas_call(
        _k5_body, grid=(G,),
        in_specs=[_row_spec((N, 512)), _row_spec((N, 129))],
        out_specs=pl.BlockSpec((512, 129), lambda i: (0, 0)),
        out_shape=jax.ShapeDtypeStruct((512, 129), jnp.float32),
    )(ass3, z)

    adj2 = pl.pallas_call(
        _k6_body, grid=(G,),
        in_specs=[_row_spec((N, 512)), _part_spec(4, 128)],
        out_specs=pl.BlockSpec((512, 512), lambda i: (0, 0)),
        out_shape=jax.ShapeDtypeStruct((512, 512), jnp.float32),
    )(ass3, s4_parts)

    z1, root, ass2, adj1 = pl.pallas_call(
        _k7_body, grid=(1,),
        in_specs=[pl.BlockSpec((512, 129), lambda i: (0, 0)),
                  pl.BlockSpec((512, 512), lambda i: (0, 0)),
                  _rspec(Wp2p), _rspec(bp2p), _rspec(sp2p),
                  _rspec(Wa2p), _rspec(ba2p), _rspec(sa2p)],
        out_specs=[pl.BlockSpec((64, 129), lambda i: (0, 0)),
                   pl.BlockSpec((1, 129), lambda i: (0, 0)),
                   pl.BlockSpec((512, 64), lambda i: (0, 0)),
                   pl.BlockSpec((64, 64), lambda i: (0, 0))],
        out_shape=[jax.ShapeDtypeStruct((64, 129), jnp.float32),
                   jax.ShapeDtypeStruct((1, 129), jnp.float32),
                   jax.ShapeDtypeStruct((512, 64), jnp.float32),
                   jax.ShapeDtypeStruct((64, 64), jnp.float32)],
    )(z2, adj2, Wp2p, bp2p, sp2p, Wa2p, ba2p, sa2p)

    ass1 = jnp.ones((64, 1), dtype=z1.dtype)
    return (z, z2, z1, root, ass3, ass2, ass1, adj2, adj1)


# speedup vs baseline: 2.5972x; 2.5093x over previous
import functools

import jax
import jax.numpy as jnp
from jax import lax
from jax.experimental import pallas as pl
from jax.experimental.pallas import tpu as pltpu
from jax.experimental.pallas import tpu_sc as plsc

N = 10000
E = 320000
TEMP = 0.2
MAX_NUMS = (512, 64)

NC, NS = 2, 16
NW = NC * NS
B = 80
EPW = E // NW
NBLK = EPW // B
NPAD = 10112
ROWS_PER_TILE = NPAD // NS


def _seg_sum_kernel(n_chunks, W):
    mesh = plsc.VectorSubcoreMesh(
        core_axis_name="c", subcore_axis_name="s", num_cores=NC, num_subcores=NS)

    @functools.partial(
        pl.kernel,
        out_type=jax.ShapeDtypeStruct((NC, n_chunks, NPAD, W), jnp.float32),
        mesh=mesh,
        scratch_types=[
            pltpu.VMEM((NBLK, B), jnp.int32),
            pltpu.VMEM((2, B), jnp.int32),
            pltpu.VMEM((2, B, W), jnp.float32),
            pltpu.VMEM_SHARED((NPAD, W), jnp.float32),
        ] + [pltpu.SemaphoreType.DMA] * 5,
        compiler_params=pltpu.CompilerParams(use_tc_tiling_on_sc=False),
    )
    def k(tab_h, src_h, dst_h, zrow_h, out_h, src_v, dst_v, rows_v, acc, *sems):
        gsems, dsems, ssem = sems[:2], sems[2:4], sems[4]
        cid = lax.axis_index("c")
        sid = lax.axis_index("s")
        wid = cid * NS + sid
        dst_page = dst_h.at[wid]
        pltpu.sync_copy(src_h.at[wid], src_v)
        for c in range(n_chunks):
            tab_c = tab_h.at[c]
            pltpu.sync_copy(zrow_h, acc.at[pl.ds(sid * ROWS_PER_TILE, ROWS_PER_TILE)])
            for s in range(2):
                pltpu.async_copy(dst_page.at[s], dst_v.at[s], dsems[s])
                pltpu.async_copy(tab_c.at[src_v.at[s]], rows_v.at[s], gsems[s])
            plsc.subcore_barrier()

            def step(b, s):
                pltpu.make_async_copy(tab_c.at[src_v.at[b]], rows_v.at[s], gsems[s]).wait()
                pltpu.make_async_copy(dst_page.at[s], dst_v.at[s], dsems[s]).wait()
                pltpu.async_copy(rows_v.at[s], acc.at[dst_v.at[s]], ssem, add=True).wait()

                @pl.when(b + 2 < NBLK)
                def _():
                    pltpu.async_copy(dst_page.at[b + 2], dst_v.at[s], dsems[s])
                    pltpu.async_copy(tab_c.at[src_v.at[b + 2]], rows_v.at[s], gsems[s])

            def pair(i, _):
                step(2 * i, 0)
                step(2 * i + 1, 1)
                return 0

            lax.fori_loop(0, NBLK // 2, pair, 0)
            step(NBLK - 1, 0)
            plsc.subcore_barrier()
            pltpu.sync_copy(
                acc.at[pl.ds(sid * ROWS_PER_TILE, ROWS_PER_TILE)],
                out_h.at[cid].at[c].at[pl.ds(sid * ROWS_PER_TILE, ROWS_PER_TILE)])
            plsc.subcore_barrier()

    return k


BLK = 2000
G = N // BLK


def _pad2(w, rows, cols):
    return jnp.pad(w, ((0, rows - w.shape[0]), (0, cols - w.shape[1])))


def _pad_lin(Wm, b, s, rows, cols):
    return _pad2(Wm, rows, cols), jnp.pad(b, (0, cols - b.shape[0]))[None, :], s.reshape(1, 1)


def _ll_block(xp, W, b, se, nonlin):
    if nonlin:
        xp = jax.nn.relu(xp)
    y = jax.lax.dot_general(xp, W, (((1,), (0,)), ((), ())),
                            preferred_element_type=jnp.float32) + b
    t = jax.nn.sigmoid(y[:, :1]) * jnp.exp(se[0, 0]) + 1.1
    xn2 = jnp.sum(y * y, axis=1, keepdims=True) - y[:, :1] * y[:, :1]
    sc = (t * t - 1.0) / jnp.clip(xn2, 1e-8, None)
    col = jax.lax.broadcasted_iota(jnp.int32, y.shape, 1)
    return jnp.where(col == 0, t, y * jnp.sqrt(sc))


def _norm_block(s):
    inner = jnp.sum(s * s, axis=1, keepdims=True) - 2.0 * s[:, :1] * s[:, :1]
    return s / jnp.sqrt(jnp.clip(jnp.abs(inner), 1e-8, None))


def _rspec(w):
    return pl.BlockSpec(w.shape, lambda i: (0,) * w.ndim)


def _k1_body(x_ref, W_ref, b_ref, s_ref, o_ref):
    xb = x_ref[...]
    nrm = jnp.clip(jnp.sqrt(jnp.sum(xb * xb, axis=1, keepdims=True)), 1e-8, None)
    ep, en = jnp.exp(nrm), jnp.exp(-nrm)
    cosh_n, sinh_n = 0.5 * (ep + en), 0.5 * (ep - en)
    h0 = jnp.concatenate(
        [cosh_n, sinh_n * xb / nrm, jnp.zeros((xb.shape[0], 15), jnp.float32)],
        axis=1)
    o_ref[...] = _ll_block(h0, W_ref[...], b_ref[...], s_ref[...], True)


def _k2_body(sab_ref, W_ref, b_ref, s_ref, o_ref):
    s_ = sab_ref[0, 0] + sab_ref[1, 0]
    o_ref[...] = _ll_block(_norm_block(s_), W_ref[...], b_ref[...], s_ref[...], True)


def _k3_body(sab_ref, Wp_ref, bp_ref, sp_ref, Wa_ref, ba_ref, sa_ref, z_ref, ll3_ref):
    s_ = sab_ref[0, 0] + sab_ref[1, 0]
    hn = _norm_block(s_)
    z_ref[...] = hn[:, :129]
    hp = _ll_block(hn, Wp_ref[...], bp_ref[...], sp_ref[...], False)
    ll3_ref[...] = _ll_block(hp, Wa_ref[...], ba_ref[...], sa_ref[...], True)


def _k4_body(sab_ref, ass_ref):
    s_ = sab_ref[0] + sab_ref[1]
    row = jnp.concatenate([s_[0], s_[1], s_[2], s_[3]], axis=1)
    nrm = _norm_block(row)
    col = jax.lax.broadcasted_iota(jnp.int32, nrm.shape, 1)
    logits = jnp.where((col >= 1) & (col <= 512), nrm / TEMP, -jnp.inf)
    m = jnp.max(logits, axis=1, keepdims=True)
    e = jnp.exp(logits - m)
    sm = e / jnp.sum(e, axis=1, keepdims=True)
    ass_ref[...] = sm[:, 1:513]


def _k5_body(ass_ref, z_ref, o_ref):
    k = pl.program_id(0)

    @pl.when(k == 0)
    def _():
        o_ref[...] = jnp.zeros_like(o_ref)

    o_ref[...] += jax.lax.dot_general(
        ass_ref[...], z_ref[...], (((0,), (0,)), ((), ())),
        preferred_element_type=jnp.float32)

    @pl.when(k == G - 1)
    def _():
        o_ref[...] = _norm_block(o_ref[...])


def _k6_body(ass_ref, sab_ref, o_ref):
    k = pl.program_id(0)

    @pl.when(k == 0)
    def _():
        o_ref[...] = jnp.zeros_like(o_ref)

    s_ = sab_ref[0] + sab_ref[1]
    aass = jnp.concatenate([s_[0], s_[1], s_[2], s_[3]], axis=1)
    o_ref[...] += jax.lax.dot_general(
        ass_ref[...], aass, (((0,), (0,)), ((), ())),
        preferred_element_type=jnp.float32)

    @pl.when(k == G - 1)
    def _():
        v = o_ref[...]
        i = jax.lax.broadcasted_iota(jnp.int32, v.shape, 0)
        j = jax.lax.broadcasted_iota(jnp.int32, v.shape, 1)
        o_ref[...] = jax.nn.sigmoid(jnp.where(i == j, 0.0, v) / TEMP)


def _k7_body(z2_ref, adj2_ref, Wp_ref, bp_ref, sp_ref, Wa_ref, ba_ref, sa_ref,
             z1_ref, root_ref, ass2_ref, adj1_ref):
    z2 = z2_ref[...]
    z2p = jnp.concatenate([z2, jnp.zeros((z2.shape[0], 15), jnp.float32)], axis=1)
    adj2 = adj2_ref[...]
    h2 = _ll_block(z2p, Wp_ref[...], bp_ref[...], sp_ref[...], False)
    h2b = _ll_block(h2, Wa_ref[...], ba_ref[...], sa_ref[...], True)
    agg2 = _norm_block(jax.lax.dot_general(
        adj2, h2b, (((1,), (0,)), ((), ())), preferred_element_type=jnp.float32))
    col = jax.lax.broadcasted_iota(jnp.int32, agg2.shape, 1)
    logits = jnp.where((col >= 1) & (col <= 64), agg2 / TEMP, -jnp.inf)
    m = jnp.max(logits, axis=1, keepdims=True)
    e = jnp.exp(logits - m)
    ass2 = (e / jnp.sum(e, axis=1, keepdims=True))[:, 1:65]
    ass2_ref[...] = ass2
    z1 = _norm_block(jax.lax.dot_general(
        ass2, z2p, (((0,), (0,)), ((), ())), preferred_element_type=jnp.float32))
    z1_ref[...] = z1[:, :129]
    root_ref[...] = _norm_block(jnp.sum(z1, axis=0, keepdims=True))[:, :129]
    a2 = jax.lax.dot_general(adj2, ass2, (((1,), (0,)), ((), ())),
                             preferred_element_type=jnp.float32)
    adj1v = jax.lax.dot_general(ass2, a2, (((0,), (0,)), ((), ())),
                                preferred_element_type=jnp.float32)
    i = jax.lax.broadcasted_iota(jnp.int32, adj1v.shape, 0)
    j = jax.lax.broadcasted_iota(jnp.int32, adj1v.shape, 1)
    adj1_ref[...] = jax.nn.sigmoid(jnp.where(i == j, 0.0, adj1v) / TEMP)


def _row_spec(w):
    return pl.BlockSpec((BLK,) + w[1:], lambda i: (i,) + (0,) * len(w[1:]))


def _part_spec(n_chunks, w):
    return pl.BlockSpec((2, n_chunks, BLK, w), lambda i: (0, 0, i, 0))


def kernel(x, edge_index, W1, b1, s1, W2, b2, s2, Wp1, bp1, sp1, Wa1, ba1, sa1, Wp2, bp2, sp2, Wa2, ba2, sa2):
    src1d = edge_index[0].reshape(NW, NBLK, B)
    dst1d = edge_index[1].reshape(NW, NBLK, B)
    zrows = {128: jnp.zeros((ROWS_PER_TILE, 128), jnp.float32),
             144: jnp.zeros((ROWS_PER_TILE, 144), jnp.float32)}

    W1p, b1p, s1p = _pad_lin(W1, b1, s1, 144, 144)
    W2p, b2p, s2p = _pad_lin(W2, b2, s2, 144, 144)
    Wp1p, bp1p, sp1p = _pad_lin(Wp1, bp1, sp1, 144, 144)
    Wa1p, ba1p, sa1p = _pad_lin(Wa1, ba1, sa1, 144, 576)
    Wp2p, bp2p, sp2p = _pad_lin(Wp2, bp2, sp2, 144, 144)
    Wa2p, ba2p, sa2p = _pad_lin(Wa2, ba2, sa2, 144, 80)

    ll1 = pl.pallas_call(
        _k1_body, grid=(G,),
        in_specs=[_row_spec((N, 128)), _rspec(W1p), _rspec(b1p), _rspec(s1p)],
        out_specs=_row_spec((N, 144)),
        out_shape=jax.ShapeDtypeStruct((N, 144), jnp.float32),
    )(x, W1p, b1p, s1p)
    s1_parts = _seg_sum_kernel(1, 144)(ll1[None], src1d, dst1d, zrows[144])

    ll2 = pl.pallas_call(
        _k2_body, grid=(G,),
        in_specs=[_part_spec(1, 144), _rspec(W2p), _rspec(b2p), _rspec(s2p)],
        out_specs=_row_spec((N, 144)),
        out_shape=jax.ShapeDtypeStruct((N, 144), jnp.float32),
    )(s1_parts, W2p, b2p, s2p)
    s2_parts = _seg_sum_kernel(1, 144)(ll2[None], src1d, dst1d, zrows[144])

    z, ll3 = pl.pallas_call(
        _k3_body, grid=(G,),
        in_specs=[_part_spec(1, 144), _rspec(Wp1p), _rspec(bp1p), _rspec(sp1p),
                  _rspec(Wa1p), _rspec(ba1p), _rspec(sa1p)],
        out_specs=[_row_spec((N, 129)), _row_spec((N, 576))],
        out_shape=[jax.ShapeDtypeStruct((N, 129), jnp.float32),
                   jax.ShapeDtypeStruct((N, 576), jnp.float32)],
    )(s2_parts, Wp1p, bp1p, sp1p, Wa1p, ba1p, sa1p)
    tab3 = ll3.reshape(N, 4, 144).transpose(1, 0, 2)
    s3_parts = _seg_sum_kernel(4, 144)(tab3, src1d, dst1d, zrows[144])

    ass3 = pl.pallas_call(
        _k4_body, grid=(G,),
        in_specs=[_part_spec(4, 144)],
        out_specs=_row_spec((N, 512)),
        out_shape=jax.ShapeDtypeStruct((N, 512), jnp.float32),
    )(s3_parts)
    tab4 = ass3.reshape(N, 4, 128).transpose(1, 0, 2)
    s4_parts = _seg_sum_kernel(4, 128)(tab4, src1d, dst1d, zrows[128])

    z2 = pl.pallas_call(
        _k5_body, grid=(G,),
        in_specs=[_row_spec((N, 512)), _row_spec((N, 129))],
        out_specs=pl.BlockSpec((512, 129), lambda i: (0, 0)),
        out_shape=jax.ShapeDtypeStruct((512, 129), jnp.float32),
    )(ass3, z)

    adj2 = pl.pallas_call(
        _k6_body, grid=(G,),
        in_specs=[_row_spec((N, 512)), _part_spec(4, 128)],
        out_specs=pl.BlockSpec((512, 512), lambda i: (0, 0)),
        out_shape=jax.ShapeDtypeStruct((512, 512), jnp.float32),
    )(ass3, s4_parts)

    z1, root, ass2, adj1 = pl.pallas_call(
        _k7_body, grid=(1,),
        in_specs=[pl.BlockSpec((512, 129), lambda i: (0, 0)),
                  pl.BlockSpec((512, 512), lambda i: (0, 0)),
                  _rspec(Wp2p), _rspec(bp2p), _rspec(sp2p),
                  _rspec(Wa2p), _rspec(ba2p), _rspec(sa2p)],
        out_specs=[pl.BlockSpec((64, 129), lambda i: (0, 0)),
                   pl.BlockSpec((1, 129), lambda i: (0, 0)),
                   pl.BlockSpec((512, 64), lambda i: (0, 0)),
                   pl.BlockSpec((64, 64), lambda i: (0, 0))],
        out_shape=[jax.ShapeDtypeStruct((64, 129), jnp.float32),
                   jax.ShapeDtypeStruct((1, 129), jnp.float32),
                   jax.ShapeDtypeStruct((512, 64), jnp.float32),
                   jax.ShapeDtypeStruct((64, 64), jnp.float32)],
    )(z2, adj2, Wp2p, bp2p, sp2p, Wa2p, ba2p, sa2p)

    ass1 = jnp.ones((64, 1), dtype=z1.dtype)
    return (z, z2, z1, root, ass3, ass2, ass1, adj2, adj1)
